# baseline (speedup 1.0000x reference)
import functools

import jax
import jax.numpy as jnp
from jax import lax
from jax.experimental import pallas as pl
from jax.experimental.pallas import tpu as pltpu
from jax.experimental.pallas import tpu_sc as plsc

N = 10000
M = 256
K = 16
OUT = 512

NC, NS = 2, 16
NW = NC * NS
CHUNK = 160
PER_W = 320
N_PAD = NW * PER_W
DV = M // 16


C3 = 8
NCHUNK = PER_W // C3


def _sc_body(x_hbm, idxf_hbm, h_hbm,
             idx0, idx1, buf0, buf1, out0, out1,
             gsem0, gsem1, wsem):
    wid = lax.axis_index("s") * NC + lax.axis_index("c")
    base = wid * PER_W
    idxs = (idx0, idx1)
    bufs = (buf0, buf1)
    outs = (out0, out1)
    gsems = (gsem0, gsem1)

    def start_gather(chunk, b):
        nb = base + chunk * C3
        pltpu.sync_copy(idxf_hbm.at[pl.ds(nb * K, C3 * K)], idxs[b])
        pltpu.async_copy(x_hbm.at[idxs[b]], bufs[b], gsems[b])

    def wait_gather(b):
        pltpu.make_async_copy(x_hbm.at[idxs[b]], bufs[b], gsems[b]).wait()

    start_gather(0, 0)

    @pl.loop(0, NCHUNK, step=2)
    def _chunks(g):
        for b in range(2):
            chunk = g + b
            buf, out = bufs[b], outs[b]
            wait_gather(b)
            @pl.when(chunk + 1 < NCHUNK)
            def _():
                start_gather(chunk + 1, 1 - b)
            @pl.when(chunk >= 2)
            def _():
                pltpu.make_async_copy(out, h_hbm.at[pl.ds(0, C3)], wsem).wait()

            @plsc.parallel_loop(0, C3 * DV)
            def _red(i):
                c = i // DV
                sl = pl.ds((i % DV) * 16, 16)
                rb = c * K
                vals = [buf[rb + k, sl] for k in range(K)]
                while len(vals) > 1:
                    vals = [vals[j] + vals[j + 1]
                            for j in range(0, len(vals) - 1, 2)] + (
                                [vals[-1]] if len(vals) % 2 else [])
                out[c, sl] = vals[0]

            pltpu.async_copy(out, h_hbm.at[pl.ds(base + chunk * C3, C3)], wsem)

    pltpu.make_async_copy(out0, h_hbm.at[pl.ds(0, C3)], wsem).wait()
    pltpu.make_async_copy(out1, h_hbm.at[pl.ds(0, C3)], wsem).wait()


def _neighbor_sum(x, idxf_pad):
    kfn = pl.kernel(
        _sc_body,
        out_type=jax.ShapeDtypeStruct((N_PAD, M), jnp.float32),
        mesh=plsc.VectorSubcoreMesh(core_axis_name="c", subcore_axis_name="s"),
        scratch_types=[
            pltpu.VMEM((C3 * K,), jnp.int32),
            pltpu.VMEM((C3 * K,), jnp.int32),
            pltpu.VMEM((C3 * K, M), jnp.float32),
            pltpu.VMEM((C3 * K, M), jnp.float32),
            pltpu.VMEM((C3, M), jnp.float32),
            pltpu.VMEM((C3, M), jnp.float32),
            pltpu.SemaphoreType.DMA,
            pltpu.SemaphoreType.DMA,
            pltpu.SemaphoreType.DMA,
        ],
    )
    return kfn(x, idxf_pad)


ROWS = 1000
GRID = N // ROWS


def _tc_matmul_body(x_ref, hnb_ref, wt_ref, b_ref, y_ref, s_ref, s2_ref):
    i = pl.program_id(0)
    h = x_ref[...] + hnb_ref[...]
    y = jnp.dot(h, wt_ref[...], preferred_element_type=jnp.float32) + b_ref[...]
    y_ref[...] = y
    s = jnp.sum(y, axis=0, keepdims=True)
    s2 = jnp.sum(y * y, axis=0, keepdims=True)

    @pl.when(i == 0)
    def _():
        s_ref[...] = s
        s2_ref[...] = s2

    @pl.when(i > 0)
    def _():
        s_ref[...] += s
        s2_ref[...] += s2


def _tc_bn_body(y_ref, s_ref, s2_ref, g_ref, beta_ref, o_ref):
    mean = s_ref[...] * (1.0 / N)
    var = s2_ref[...] * (1.0 / N) - mean * mean
    scale = g_ref[...] * lax.rsqrt(var + 1e-5)
    shift = beta_ref[...] - mean * scale
    o_ref[...] = jnp.maximum(y_ref[...] * scale + shift, 0.0)


def kernel(nodes_features, nodes_neighbors_indexes, W, b, gamma, beta):
    x = nodes_features
    idxf_pad = jnp.pad(nodes_neighbors_indexes.reshape(-1),
                       (0, (N_PAD - N) * K))
    hnb = _neighbor_sum(x, idxf_pad)[:N]

    wt = W.T
    b2 = b.reshape(1, OUT)
    g2 = gamma.reshape(1, OUT)
    beta2 = beta.reshape(1, OUT)

    y, s, s2 = pl.pallas_call(
        _tc_matmul_body,
        grid=(GRID,),
        in_specs=[
            pl.BlockSpec((ROWS, M), lambda i: (i, 0)),
            pl.BlockSpec((ROWS, M), lambda i: (i, 0)),
            pl.BlockSpec((M, OUT), lambda i: (0, 0)),
            pl.BlockSpec((1, OUT), lambda i: (0, 0)),
        ],
        out_specs=[
            pl.BlockSpec((ROWS, OUT), lambda i: (i, 0)),
            pl.BlockSpec((1, OUT), lambda i: (0, 0)),
            pl.BlockSpec((1, OUT), lambda i: (0, 0)),
        ],
        out_shape=[
            jax.ShapeDtypeStruct((N, OUT), jnp.float32),
            jax.ShapeDtypeStruct((1, OUT), jnp.float32),
            jax.ShapeDtypeStruct((1, OUT), jnp.float32),
        ],
    )(x, hnb, wt, b2)

    out = pl.pallas_call(
        _tc_bn_body,
        grid=(GRID,),
        in_specs=[
            pl.BlockSpec((ROWS, OUT), lambda i: (i, 0)),
            pl.BlockSpec((1, OUT), lambda i: (0, 0)),
            pl.BlockSpec((1, OUT), lambda i: (0, 0)),
            pl.BlockSpec((1, OUT), lambda i: (0, 0)),
            pl.BlockSpec((1, OUT), lambda i: (0, 0)),
        ],
        out_specs=pl.BlockSpec((ROWS, OUT), lambda i: (i, 0)),
        out_shape=jax.ShapeDtypeStruct((N, OUT), jnp.float32),
    )(y, s, s2, g2, beta2)

    return (out, nodes_neighbors_indexes)


# speedup vs baseline: 1.0734x; 1.0734x over previous
import functools

import jax
import jax.numpy as jnp
from jax import lax
from jax.experimental import pallas as pl
from jax.experimental.pallas import tpu as pltpu
from jax.experimental.pallas import tpu_sc as plsc

N = 10000
M = 256
K = 16
OUT = 512

NC, NS = 2, 16
NW = NC * NS
CHUNK = 160
PER_W = 320
N_PAD = NW * PER_W
DV = M // 16


C3 = 8
NCHUNK = PER_W // C3


NBUF = 3
WBATCH = 4


def _sc_body(x_hbm, idxf_hbm, h_hbm,
             idx_all, buf0, buf1, buf2, out0, out1,
             gsem0, gsem1, gsem2, wsem0, wsem1):
    wid = lax.axis_index("s") * NC + lax.axis_index("c")
    base = wid * PER_W
    bufs = (buf0, buf1, buf2)
    gsems = (gsem0, gsem1, gsem2)
    outs = (out0, out1)
    wsems = (wsem0, wsem1)

    pltpu.sync_copy(idxf_hbm.at[pl.ds(base * K, PER_W * K)], idx_all)

    def gather_desc(chunk):
        b = chunk % NBUF
        return pltpu.make_async_copy(
            x_hbm.at[idx_all.at[pl.ds(chunk * C3 * K, C3 * K)]],
            bufs[b], gsems[b])

    for c in range(NBUF):
        gather_desc(c).start()

    for chunk in range(NCHUNK):
        buf = bufs[chunk % NBUF]
        grp = chunk // WBATCH
        out = outs[grp % 2]
        orow = (chunk % WBATCH) * C3
        gather_desc(chunk).wait()
        if chunk % WBATCH == 0 and grp >= 2:
            pltpu.make_async_copy(
                out, h_hbm.at[pl.ds(0, WBATCH * C3)], wsems[grp % 2]).wait()

        @plsc.parallel_loop(0, C3 * DV)
        def _red(i):
            c = i // DV
            sl = pl.ds((i % DV) * 16, 16)
            rb = c * K
            vals = [buf[rb + k, sl] for k in range(K)]
            while len(vals) > 1:
                vals = [vals[j] + vals[j + 1]
                        for j in range(0, len(vals) - 1, 2)] + (
                            [vals[-1]] if len(vals) % 2 else [])
            out[orow + c, sl] = vals[0]

        if chunk + NBUF < NCHUNK:
            gather_desc(chunk + NBUF).start()
        if chunk % WBATCH == WBATCH - 1:
            pltpu.async_copy(
                out, h_hbm.at[pl.ds(base + grp * WBATCH * C3, WBATCH * C3)],
                wsems[grp % 2])

    for g in (0, 1):
        pltpu.make_async_copy(
            outs[g], h_hbm.at[pl.ds(0, WBATCH * C3)], wsems[g]).wait()


def _neighbor_sum(x, idxf_pad):
    kfn = pl.kernel(
        _sc_body,
        out_type=jax.ShapeDtypeStruct((N_PAD, M), jnp.float32),
        mesh=plsc.VectorSubcoreMesh(core_axis_name="c", subcore_axis_name="s"),
        scratch_types=[
            pltpu.VMEM((PER_W * K,), jnp.int32),
            pltpu.VMEM((C3 * K, M), jnp.float32),
            pltpu.VMEM((C3 * K, M), jnp.float32),
            pltpu.VMEM((C3 * K, M), jnp.float32),
            pltpu.VMEM((WBATCH * C3, M), jnp.float32),
            pltpu.VMEM((WBATCH * C3, M), jnp.float32),
            pltpu.SemaphoreType.DMA,
            pltpu.SemaphoreType.DMA,
            pltpu.SemaphoreType.DMA,
            pltpu.SemaphoreType.DMA,
            pltpu.SemaphoreType.DMA,
        ],
    )
    return kfn(x, idxf_pad)


ROWS = 1000
GRID = N // ROWS


def _tc_matmul_body(x_ref, hnb_ref, wt_ref, b_ref, y_ref, s_ref, s2_ref):
    i = pl.program_id(0)
    h = x_ref[...] + hnb_ref[...]
    y = jnp.dot(h, wt_ref[...], preferred_element_type=jnp.float32) + b_ref[...]
    y_ref[...] = y
    s = jnp.sum(y, axis=0, keepdims=True)
    s2 = jnp.sum(y * y, axis=0, keepdims=True)

    @pl.when(i == 0)
    def _():
        s_ref[...] = s
        s2_ref[...] = s2

    @pl.when(i > 0)
    def _():
        s_ref[...] += s
        s2_ref[...] += s2


def _tc_bn_body(y_ref, s_ref, s2_ref, g_ref, beta_ref, o_ref):
    mean = s_ref[...] * (1.0 / N)
    var = s2_ref[...] * (1.0 / N) - mean * mean
    scale = g_ref[...] * lax.rsqrt(var + 1e-5)
    shift = beta_ref[...] - mean * scale
    o_ref[...] = jnp.maximum(y_ref[...] * scale + shift, 0.0)


def kernel(nodes_features, nodes_neighbors_indexes, W, b, gamma, beta):
    x = nodes_features
    idxf_pad = jnp.pad(nodes_neighbors_indexes.reshape(-1),
                       (0, (N_PAD - N) * K))
    hnb = _neighbor_sum(x, idxf_pad)[:N]

    wt = W.T
    b2 = b.reshape(1, OUT)
    g2 = gamma.reshape(1, OUT)
    beta2 = beta.reshape(1, OUT)

    y, s, s2 = pl.pallas_call(
        _tc_matmul_body,
        grid=(GRID,),
        in_specs=[
            pl.BlockSpec((ROWS, M), lambda i: (i, 0)),
            pl.BlockSpec((ROWS, M), lambda i: (i, 0)),
            pl.BlockSpec((M, OUT), lambda i: (0, 0)),
            pl.BlockSpec((1, OUT), lambda i: (0, 0)),
        ],
        out_specs=[
            pl.BlockSpec((ROWS, OUT), lambda i: (i, 0)),
            pl.BlockSpec((1, OUT), lambda i: (0, 0)),
            pl.BlockSpec((1, OUT), lambda i: (0, 0)),
        ],
        out_shape=[
            jax.ShapeDtypeStruct((N, OUT), jnp.float32),
            jax.ShapeDtypeStruct((1, OUT), jnp.float32),
            jax.ShapeDtypeStruct((1, OUT), jnp.float32),
        ],
    )(x, hnb, wt, b2)

    out = pl.pallas_call(
        _tc_bn_body,
        grid=(GRID,),
        in_specs=[
            pl.BlockSpec((ROWS, OUT), lambda i: (i, 0)),
            pl.BlockSpec((1, OUT), lambda i: (0, 0)),
            pl.BlockSpec((1, OUT), lambda i: (0, 0)),
            pl.BlockSpec((1, OUT), lambda i: (0, 0)),
            pl.BlockSpec((1, OUT), lambda i: (0, 0)),
        ],
        out_specs=pl.BlockSpec((ROWS, OUT), lambda i: (i, 0)),
        out_shape=jax.ShapeDtypeStruct((N, OUT), jnp.float32),
    )(y, s, s2, g2, beta2)

    return (out, nodes_neighbors_indexes)


# speedup vs baseline: 1.1232x; 1.0463x over previous
import functools

import jax
import jax.numpy as jnp
from jax import lax
from jax.experimental import pallas as pl
from jax.experimental.pallas import tpu as pltpu
from jax.experimental.pallas import tpu_sc as plsc

N = 10000
M = 256
K = 16
OUT = 512

NC, NS = 2, 16
NW = NC * NS
CHUNK = 160
PER_W = 320
N_PAD = NW * PER_W
DV = M // 16


C3 = 8

FAST_CORE = 0
FW, SW = 512, 128
FAST_TOT = NS * FW


def _sc_body(x_hbm, idxf_hbm, h_hbm,
             idx_all, buf0, buf1, out_v, gsem0, gsem1, wsem):
    c = lax.axis_index("c")
    s = lax.axis_index("s")
    is_fast = c == FAST_CORE
    base = jnp.where(is_fast, s * FW, FAST_TOT + s * SW)
    nchunk = jnp.where(is_fast, FW // C3, SW // C3)
    nrounds = nchunk // 2
    bufs = (buf0, buf1)
    gsems = (gsem0, gsem1)

    @pl.when(is_fast)
    def _():
        pltpu.sync_copy(idxf_hbm.at[pl.ds(base * K, FW * K)], idx_all)

    @pl.when(jnp.logical_not(is_fast))
    def _():
        pltpu.sync_copy(idxf_hbm.at[pl.ds(base * K, SW * K)],
                        idx_all.at[pl.ds(0, SW * K)])

    def gdesc(chunk, b):
        return pltpu.make_async_copy(
            x_hbm.at[idx_all.at[pl.ds(chunk * (C3 * K), C3 * K)]],
            bufs[b], gsems[b])

    gdesc(0, 0).start()
    gdesc(1, 1).start()

    @pl.loop(0, nrounds)
    def _round(g):
        @pl.when(g > 0)
        def _():
            pltpu.make_async_copy(out_v, h_hbm.at[pl.ds(0, 2 * C3)],
                                  wsem).wait()
        for b in range(2):
            chunk = 2 * g + b
            buf = bufs[b]
            gdesc(chunk, b).wait()

            @plsc.parallel_loop(0, C3 * DV)
            def _red(i):
                cc = i // DV
                sl = pl.ds((i % DV) * 16, 16)
                rb = cc * K
                vals = [buf[rb + k, sl] for k in range(K)]
                while len(vals) > 1:
                    vals = [vals[j] + vals[j + 1]
                            for j in range(0, len(vals) - 1, 2)] + (
                                [vals[-1]] if len(vals) % 2 else [])
                out_v[b * C3 + cc, sl] = vals[0]

            @pl.when(chunk + 2 < nchunk)
            def _():
                gdesc(chunk + 2, b).start()

        pltpu.async_copy(out_v, h_hbm.at[pl.ds(base + g * (2 * C3), 2 * C3)],
                         wsem)

    pltpu.make_async_copy(out_v, h_hbm.at[pl.ds(0, 2 * C3)], wsem).wait()


def _neighbor_sum(x, idxf_pad):
    kfn = pl.kernel(
        _sc_body,
        out_type=jax.ShapeDtypeStruct((N_PAD, M), jnp.float32),
        mesh=plsc.VectorSubcoreMesh(core_axis_name="c", subcore_axis_name="s"),
        scratch_types=[
            pltpu.VMEM((FW * K,), jnp.int32),
            pltpu.VMEM((C3 * K, M), jnp.float32),
            pltpu.VMEM((C3 * K, M), jnp.float32),
            pltpu.VMEM((2 * C3, M), jnp.float32),
            pltpu.SemaphoreType.DMA,
            pltpu.SemaphoreType.DMA,
            pltpu.SemaphoreType.DMA,
        ],
    )
    return kfn(x, idxf_pad)


ROWS = 1000
GRID = N // ROWS


def _tc_matmul_body(x_ref, hnb_ref, wt_ref, b_ref, y_ref, s_ref, s2_ref):
    i = pl.program_id(0)
    h = x_ref[...] + hnb_ref[...]
    y = jnp.dot(h, wt_ref[...], preferred_element_type=jnp.float32) + b_ref[...]
    y_ref[...] = y
    s = jnp.sum(y, axis=0, keepdims=True)
    s2 = jnp.sum(y * y, axis=0, keepdims=True)

    @pl.when(i == 0)
    def _():
        s_ref[...] = s
        s2_ref[...] = s2

    @pl.when(i > 0)
    def _():
        s_ref[...] += s
        s2_ref[...] += s2


def _tc_bn_body(y_ref, s_ref, s2_ref, g_ref, beta_ref, o_ref):
    mean = s_ref[...] * (1.0 / N)
    var = s2_ref[...] * (1.0 / N) - mean * mean
    scale = g_ref[...] * lax.rsqrt(var + 1e-5)
    shift = beta_ref[...] - mean * scale
    o_ref[...] = jnp.maximum(y_ref[...] * scale + shift, 0.0)


def kernel(nodes_features, nodes_neighbors_indexes, W, b, gamma, beta):
    x = nodes_features
    idxf_pad = jnp.pad(nodes_neighbors_indexes.reshape(-1),
                       (0, (N_PAD - N) * K))
    hnb = _neighbor_sum(x, idxf_pad)[:N]

    wt = W.T
    b2 = b.reshape(1, OUT)
    g2 = gamma.reshape(1, OUT)
    beta2 = beta.reshape(1, OUT)

    y, s, s2 = pl.pallas_call(
        _tc_matmul_body,
        grid=(GRID,),
        in_specs=[
            pl.BlockSpec((ROWS, M), lambda i: (i, 0)),
            pl.BlockSpec((ROWS, M), lambda i: (i, 0)),
            pl.BlockSpec((M, OUT), lambda i: (0, 0)),
            pl.BlockSpec((1, OUT), lambda i: (0, 0)),
        ],
        out_specs=[
            pl.BlockSpec((ROWS, OUT), lambda i: (i, 0)),
            pl.BlockSpec((1, OUT), lambda i: (0, 0)),
            pl.BlockSpec((1, OUT), lambda i: (0, 0)),
        ],
        out_shape=[
            jax.ShapeDtypeStruct((N, OUT), jnp.float32),
            jax.ShapeDtypeStruct((1, OUT), jnp.float32),
            jax.ShapeDtypeStruct((1, OUT), jnp.float32),
        ],
    )(x, hnb, wt, b2)

    out = pl.pallas_call(
        _tc_bn_body,
        grid=(GRID,),
        in_specs=[
            pl.BlockSpec((ROWS, OUT), lambda i: (i, 0)),
            pl.BlockSpec((1, OUT), lambda i: (0, 0)),
            pl.BlockSpec((1, OUT), lambda i: (0, 0)),
            pl.BlockSpec((1, OUT), lambda i: (0, 0)),
            pl.BlockSpec((1, OUT), lambda i: (0, 0)),
        ],
        out_specs=pl.BlockSpec((ROWS, OUT), lambda i: (i, 0)),
        out_shape=jax.ShapeDtypeStruct((N, OUT), jnp.float32),
    )(y, s, s2, g2, beta2)

    return (out, nodes_neighbors_indexes)
